# baseline (speedup 1.0000x reference)
import functools

import jax
import jax.numpy as jnp
from jax import lax
from jax.experimental import pallas as pl
from jax.experimental.pallas import tpu as pltpu
from jax.experimental.pallas import tpu_sc as plsc

NUM_EMB = 20000
D = 512
BATCH = 4096
HIST = 50
B = BATCH * HIST

_info = plsc.get_sparse_core_info()
_NC, _NS = _info.num_cores, _info.num_subcores
NW = _NC * _NS
B_PER_W = B // NW
CHUNK = 40
NCHUNK = B_PER_W // CHUNK
NBUF = 4
NGROUP = NCHUNK // NBUF


def _emb_body(table_hbm, idx_hbm, out_hbm, idx_v,
              r0, r1, r2, r3, sg0, sg1, sg2, sg3, sw0, sw1, sw2, sw3):
    wid = lax.axis_index("s") * _NC + lax.axis_index("c")
    base = wid * B_PER_W
    rows = (r0, r1, r2, r3)
    sgs = (sg0, sg1, sg2, sg3)
    sws = (sw0, sw1, sw2, sw3)
    pltpu.sync_copy(idx_hbm.at[pl.ds(base, B_PER_W)], idx_v)

    def start_g(b, c):
        pltpu.async_copy(
            table_hbm.at[idx_v.at[pl.ds(c * CHUNK, CHUNK)]], rows[b], sgs[b])

    def wait_g(b, c):
        pltpu.make_async_copy(
            table_hbm.at[idx_v.at[pl.ds(c * CHUNK, CHUNK)]], rows[b], sgs[b]).wait()

    def start_w(b, c):
        pltpu.async_copy(
            rows[b], out_hbm.at[pl.ds(base + c * CHUNK, CHUNK)], sws[b])

    def wait_w(b, c):
        pltpu.make_async_copy(
            rows[b], out_hbm.at[pl.ds(base + c * CHUNK, CHUNK)], sws[b]).wait()

    for b in range(NBUF):
        start_g(b, b)

    def outer(i, carry):
        for b in range(NBUF):
            c = NBUF * i + b
            wait_g(b, c)
            start_w(b, c)
        for b in range(NBUF):
            c = NBUF * i + b
            wait_w(b, c)
            start_g(b, c + NBUF)
        return carry

    lax.fori_loop(0, NGROUP - 1, outer, 0)

    for b in range(NBUF):
        c = NCHUNK - NBUF + b
        wait_g(b, c)
        start_w(b, c)
    for b in range(NBUF):
        c = NCHUNK - NBUF + b
        wait_w(b, c)


@jax.jit
def _emb(table, idx_flat):
    run = pl.kernel(
        _emb_body,
        out_type=jax.ShapeDtypeStruct((B, D), jnp.float32),
        mesh=plsc.VectorSubcoreMesh(core_axis_name="c", subcore_axis_name="s"),
        scratch_types=(
            [pltpu.VMEM((B_PER_W,), jnp.int32)]
            + [pltpu.VMEM((CHUNK, D), jnp.float32)] * NBUF
            + [pltpu.SemaphoreType.DMA] * (2 * NBUF)
        ),
    )
    return run(table, idx_flat)


def kernel(indices, weight):
    idx_flat = indices.astype(jnp.int32).T.reshape(-1)
    flat = _emb(weight, idx_flat)
    return flat.reshape(HIST, BATCH, D).transpose(1, 0, 2)


# speedup vs baseline: 1.0056x; 1.0056x over previous
import functools

import jax
import jax.numpy as jnp
from jax import lax
from jax.experimental import pallas as pl
from jax.experimental.pallas import tpu as pltpu
from jax.experimental.pallas import tpu_sc as plsc

NUM_EMB = 20000
D = 512
BATCH = 4096
HIST = 50
B = BATCH * HIST

_info = plsc.get_sparse_core_info()
_NC, _NS = _info.num_cores, _info.num_subcores
NW = _NC * _NS
B_PER_W = B // NW
CHUNK = 80
NCHUNK = B_PER_W // CHUNK
NPAIR = NCHUNK // 2


def _emb_body(table_hbm, idx_hbm, out_hbm, idx_v, rows0, rows1, sem0, sem1):
    wid = lax.axis_index("s") * _NC + lax.axis_index("c")
    base = wid * B_PER_W
    rows = (rows0, rows1)
    sems = (sem0, sem1)
    pltpu.sync_copy(idx_hbm.at[wid], idx_v)

    pltpu.async_copy(table_hbm.at[idx_v.at[0]], rows0, sem0)
    pltpu.async_copy(table_hbm.at[idx_v.at[1]], rows1, sem1)

    def outer(i, carry):
        for b in range(2):
            c = 2 * i + b
            pltpu.make_async_copy(table_hbm.at[idx_v.at[c]], rows[b], sems[b]).wait()
            pltpu.sync_copy(rows[b], out_hbm.at[pl.ds(base + c * CHUNK, CHUNK)])
            pltpu.async_copy(table_hbm.at[idx_v.at[c + 2]], rows[b], sems[b])
        return carry

    lax.fori_loop(0, NPAIR - 1, outer, 0)

    for b in range(2):
        c = NCHUNK - 2 + b
        pltpu.make_async_copy(table_hbm.at[idx_v.at[c]], rows[b], sems[b]).wait()
        pltpu.sync_copy(rows[b], out_hbm.at[pl.ds(base + c * CHUNK, CHUNK)])


@jax.jit
def _emb(table, idx3):
    run = pl.kernel(
        _emb_body,
        out_type=jax.ShapeDtypeStruct((B, D), jnp.float32),
        mesh=plsc.VectorSubcoreMesh(core_axis_name="c", subcore_axis_name="s"),
        scratch_types=[
            pltpu.VMEM((NCHUNK, CHUNK), jnp.int32),
            pltpu.VMEM((CHUNK, D), jnp.float32),
            pltpu.VMEM((CHUNK, D), jnp.float32),
            pltpu.SemaphoreType.DMA,
            pltpu.SemaphoreType.DMA,
        ],
    )
    return run(table, idx3)


def kernel(indices, weight):
    idx3 = indices.astype(jnp.int32).T.reshape(NW, NCHUNK, CHUNK)
    flat = _emb(weight, idx3)
    return flat.reshape(HIST, BATCH, D).transpose(1, 0, 2)
